# baseline (speedup 1.0000x reference)
import functools

import jax
import jax.numpy as jnp
from jax import lax
from jax.experimental import pallas as pl
from jax.experimental.pallas import tpu as pltpu
from jax.experimental.pallas import tpu_sc as plsc

H = 128
NB = 8
E = 160000
T = 320000

NC = 2
NSUB = 16
NW = NC * NSUB
L = 16

BE = 1280


def _edge_pre_body(x_ref, rbft_ref, wrbf_ref, wkj_ref, bkj_ref, wji_ref,
                   bji_ref, xji_ref, xkj_ref):
    xb = x_ref[...]
    ji = jnp.dot(xb, wji_ref[...], preferred_element_type=jnp.float32) + bji_ref[...]
    xji_ref[...] = ji * jax.nn.sigmoid(ji)
    kj = jnp.dot(xb, wkj_ref[...], preferred_element_type=jnp.float32) + bkj_ref[...]
    rt = jax.lax.dot_general(rbft_ref[...], wrbf_ref[...],
                             (((0,), (0,)), ((), ())),
                             preferred_element_type=jnp.float32)
    xkj_ref[...] = kj * jax.nn.sigmoid(kj) * rt


def _edge_pre(x, rbf, W_rbf, W_kj, b_kj, W_ji, b_ji):
    nb = E // BE
    return pl.pallas_call(
        _edge_pre_body,
        grid=(nb,),
        in_specs=[
            pl.BlockSpec((BE, H), lambda i: (i, 0)),
            pl.BlockSpec((6, BE), lambda i: (0, i)),
            pl.BlockSpec((6, H), lambda i: (0, 0)),
            pl.BlockSpec((H, H), lambda i: (0, 0)),
            pl.BlockSpec((1, H), lambda i: (0, 0)),
            pl.BlockSpec((H, H), lambda i: (0, 0)),
            pl.BlockSpec((1, H), lambda i: (0, 0)),
        ],
        out_specs=[
            pl.BlockSpec((BE, H), lambda i: (i, 0)),
            pl.BlockSpec((BE, H), lambda i: (i, 0)),
        ],
        out_shape=[
            jax.ShapeDtypeStruct((E, H), jnp.float32),
            jax.ShapeDtypeStruct((E, H), jnp.float32),
        ],
    )(x, rbf.T, W_rbf, W_kj, b_kj.reshape(1, H), W_ji, b_ji.reshape(1, H))


BT = 1280


def _triplet_body(g_ref, sbft_ref, wsbf_ref, wbil_ref, out_ref):
    st = jax.lax.dot_general(sbft_ref[...], wsbf_ref[...],
                             (((0,), (0,)), ((), ())),
                             preferred_element_type=jnp.float32)
    g = g_ref[...]
    acc = jnp.zeros((BT, H), jnp.float32)
    for j in range(NB):
        acc = acc + jnp.dot(g * st[:, j:j + 1], wbil_ref[j],
                            preferred_element_type=jnp.float32)
    out_ref[...] = acc


def _triplet(gathered, sbf, W_sbf, Wbil_t):
    nb = T // BT
    return pl.pallas_call(
        _triplet_body,
        grid=(nb,),
        in_specs=[
            pl.BlockSpec((BT, H), lambda i: (i, 0)),
            pl.BlockSpec((42, BT), lambda i: (0, i)),
            pl.BlockSpec((42, NB), lambda i: (0, 0)),
            pl.BlockSpec((NB, H, H), lambda i: (0, 0, 0)),
        ],
        out_specs=pl.BlockSpec((BT, H), lambda i: (i, 0)),
        out_shape=jax.ShapeDtypeStruct((T, H), jnp.float32),
    )(gathered, sbf.T, W_sbf, Wbil_t)


def _post_body(x_ref, xji_ref, seg_ref,
               w1_ref, b1_ref, w2_ref, b2_ref,
               wl_ref, bl_ref,
               w3_ref, b3_ref, w4_ref, b4_ref,
               w5_ref, b5_ref, w6_ref, b6_ref,
               out_ref):
    def mm(a, w_ref, b_ref):
        r = jnp.dot(a, w_ref[...], preferred_element_type=jnp.float32) + b_ref[...]
        return r * jax.nn.sigmoid(r)

    h = xji_ref[...] + seg_ref[...]
    h = h + mm(mm(h, w1_ref, b1_ref), w2_ref, b2_ref)
    h = mm(h, wl_ref, bl_ref) + x_ref[...]
    h = h + mm(mm(h, w3_ref, b3_ref), w4_ref, b4_ref)
    h = h + mm(mm(h, w5_ref, b5_ref), w6_ref, b6_ref)
    out_ref[...] = h


def _post(x, x_ji, seg, rb0_w1, rb0_b1, rb0_w2, rb0_b2, W_lin, b_lin,
          ra0_w1, ra0_b1, ra0_w2, ra0_b2, ra1_w1, ra1_b1, ra1_w2, ra1_b2):
    nb = E // BE
    wspec = pl.BlockSpec((H, H), lambda i: (0, 0))
    bspec = pl.BlockSpec((1, H), lambda i: (0, 0))
    return pl.pallas_call(
        _post_body,
        grid=(nb,),
        in_specs=[pl.BlockSpec((BE, H), lambda i: (i, 0))] * 3 +
                 [wspec, bspec] * 7,
        out_specs=pl.BlockSpec((BE, H), lambda i: (i, 0)),
        out_shape=jax.ShapeDtypeStruct((E, H), jnp.float32),
    )(x, x_ji, seg,
      rb0_w1, rb0_b1.reshape(1, H), rb0_w2, rb0_b2.reshape(1, H),
      W_lin, b_lin.reshape(1, H),
      ra0_w1, ra0_b1.reshape(1, H), ra0_w2, ra0_b2.reshape(1, H),
      ra1_w1, ra1_b1.reshape(1, H), ra1_w2, ra1_b2.reshape(1, H))


TPW = T // NW
GB = 80
NGB = TPW // GB


def _sc_gather_body(table_hbm, idx_hbm, out_hbm, idx_v, rows_v, gsem, osem):
    c = lax.axis_index("c")
    s = lax.axis_index("s")
    wid = s * NC + c
    base = wid * TPW
    pltpu.sync_copy(idx_hbm.at[pl.ds(base * 1, TPW)], idx_v)

    def g_desc(b, slot):
        return pltpu.make_async_copy(
            table_hbm.at[idx_v.at[pl.ds(b * GB, GB)]],
            rows_v.at[slot], gsem.at[slot])

    def o_desc(b, slot):
        return pltpu.make_async_copy(
            rows_v.at[slot], out_hbm.at[pl.ds(base + b * GB, GB)],
            osem.at[slot])

    g_desc(0, 0).start()

    def body(b, _):
        slot = lax.rem(b, 2)
        nslot = 1 - slot
        g_desc(b, slot).wait()

        @pl.when(b >= 1)
        def _():
            o_desc(b - 1, nslot).wait()

        @pl.when(b < NGB - 1)
        def _():
            g_desc(b + 1, nslot).start()

        o_desc(b, slot).start()
        return 0

    lax.fori_loop(0, NGB, body, 0)
    o_desc(NGB - 1, (NGB - 1) % 2).wait()


def _sc_gather(table, idx):
    mesh = plsc.VectorSubcoreMesh(core_axis_name="c", subcore_axis_name="s")
    f = pl.kernel(
        _sc_gather_body,
        out_type=jax.ShapeDtypeStruct((T, H), jnp.float32),
        mesh=mesh,
        scratch_types=[
            pltpu.VMEM((TPW,), jnp.int32),
            pltpu.VMEM((2, GB, H), jnp.float32),
            pltpu.SemaphoreType.DMA((2,)),
            pltpu.SemaphoreType.DMA((2,)),
        ],
    )
    return f(table, idx)


EPC = E // NC
ACC = EPC + NSUB
TPS = T // NSUB
SB = 80
RB = 800
NRB = TPS // RB
SPB = RB // SB
NH = H // L
ZR = ACC // NSUB
ZC = 125
WR = EPC // NSUB
IC = 2000
NIC = TPS // IC


def _sc_scatter_body(outt_hbm, idx_hbm, seg_hbm, idxc, rel2d, rows_v, zbuf,
                     acc, rsem, ssem):
    c = lax.axis_index("c")
    s = lax.axis_index("s")
    t0 = s * TPS
    lo = c * EPC
    trash = EPC + s

    def zb(i, _):
        zbuf[i, :] = jnp.zeros((L,), jnp.float32)
        return 0
    lax.fori_loop(0, ZC, zb, 0)

    def relchunk(ci, _):
        pltpu.sync_copy(idx_hbm.at[pl.ds(t0 + ci * IC, IC)], idxc)

        def relbody(i, _):
            v = idxc[pl.ds(i * L, L)]
            rel = v - lo
            inb = (rel >= 0) & (rel < EPC)
            relv = jnp.where(inb, rel, trash)
            rel2d[ci * (IC // SB) + lax.div(i, 5),
                  pl.ds(lax.rem(i, 5) * L, L)] = relv
            return 0
        lax.fori_loop(0, IC // L, relbody, 0)
        return 0
    lax.fori_loop(0, NIC, relchunk, 0)

    def r_desc(rb, slot, h):
        return pltpu.make_async_copy(
            outt_hbm.at[pl.ds(t0 + rb * RB, RB), pl.ds(h * L, L)],
            rows_v.at[slot], rsem.at[slot])

    def s_desc(rb, k, slot):
        return pltpu.make_async_copy(
            rows_v.at[slot, pl.ds(k * SB, SB)],
            acc.at[rel2d.at[rb * SPB + k]], ssem.at[slot])

    def one_pass(h, _):
        for z in range(40):
            pltpu.sync_copy(zbuf, acc.at[pl.ds(s * ZR + z * ZC, ZC)])
        pltpu.sync_copy(zbuf.at[pl.ds(0, 1)], acc.at[pl.ds(s * ZR + 40 * ZC, 1)])
        plsc.subcore_barrier()

        r_desc(0, 0, h).start()

        def body(rb, _):
            slot = lax.rem(rb, 2)
            nslot = 1 - slot
            r_desc(rb, slot, h).wait()

            @pl.when(rb < NRB - 1)
            def _():
                r_desc(rb + 1, nslot, h).start()

            for k in range(SPB):
                pltpu.async_copy(rows_v.at[slot, pl.ds(k * SB, SB)],
                                 acc.at[rel2d.at[rb * SPB + k]],
                                 ssem.at[slot], add=True)
            for k in range(SPB):
                s_desc(rb, k, slot).wait()
            return 0

        lax.fori_loop(0, NRB, body, 0)
        plsc.subcore_barrier()

        pltpu.sync_copy(
            acc.at[pl.ds(s * WR, WR)],
            seg_hbm.at[pl.ds(lo + s * WR, WR), pl.ds(h * L, L)])
        plsc.subcore_barrier()
        return 0

    lax.fori_loop(0, NH, one_pass, 0)


def _sc_scatter(out_t, idx):
    mesh = plsc.VectorSubcoreMesh(core_axis_name="c", subcore_axis_name="s")
    f = pl.kernel(
        _sc_scatter_body,
        out_type=jax.ShapeDtypeStruct((E, H), jnp.float32),
        mesh=mesh,
        compiler_params=pltpu.CompilerParams(use_tc_tiling_on_sc=False),
        scratch_types=[
            pltpu.VMEM((IC,), jnp.int32),
            pltpu.VMEM((TPS // SB, SB), jnp.int32),
            pltpu.VMEM((2, RB, L), jnp.float32),
            pltpu.VMEM((ZC, L), jnp.float32),
            pltpu.MemorySpace.VMEM_SHARED((ACC, L), jnp.float32),
            pltpu.SemaphoreType.DMA((2,)),
            pltpu.SemaphoreType.DMA((2,)),
        ],
    )
    return f(out_t, idx)


def kernel(x, rbf, sbf, idx_kj, idx_ji, W_rbf, W_sbf, W_kj, b_kj, W_ji, b_ji,
           Wbil, rb0_w1, rb0_b1, rb0_w2, rb0_b2, W_lin, b_lin,
           ra0_w1, ra0_b1, ra0_w2, ra0_b2, ra1_w1, ra1_b1, ra1_w2, ra1_b2):
    x_ji, x_kj2 = _edge_pre(x, rbf, W_rbf, W_kj, b_kj, W_ji, b_ji)
    gathered = _sc_gather(x_kj2, idx_kj)
    Wbil_t = jnp.transpose(Wbil, (1, 2, 0))
    out_t = _triplet(gathered, sbf, W_sbf, Wbil_t)
    seg = _sc_scatter(out_t, idx_ji)
    return _post(x, x_ji, seg, rb0_w1, rb0_b1, rb0_w2, rb0_b2, W_lin, b_lin,
                 ra0_w1, ra0_b1, ra0_w2, ra0_b2, ra1_w1, ra1_b1, ra1_w2, ra1_b2)


# speedup vs baseline: 1.0981x; 1.0981x over previous
import functools

import jax
import jax.numpy as jnp
from jax import lax
from jax.experimental import pallas as pl
from jax.experimental.pallas import tpu as pltpu
from jax.experimental.pallas import tpu_sc as plsc

H = 128
NB = 8
E = 160000
T = 320000

NC = 2
NSUB = 16
NW = NC * NSUB
L = 16

BE = 3200


def _edge_pre_body(x_ref, rbft_ref, wrbf_ref, wkj_ref, bkj_ref, wji_ref,
                   bji_ref, xji_ref, xkj_ref):
    xb = x_ref[...]
    ji = jnp.dot(xb, wji_ref[...], preferred_element_type=jnp.float32) + bji_ref[...]
    xji_ref[...] = ji * jax.nn.sigmoid(ji)
    kj = jnp.dot(xb, wkj_ref[...], preferred_element_type=jnp.float32) + bkj_ref[...]
    rt = jax.lax.dot_general(rbft_ref[...], wrbf_ref[...],
                             (((0,), (0,)), ((), ())),
                             preferred_element_type=jnp.float32)
    xkj_ref[...] = kj * jax.nn.sigmoid(kj) * rt


def _edge_pre(x, rbf, W_rbf, W_kj, b_kj, W_ji, b_ji):
    nb = E // BE
    return pl.pallas_call(
        _edge_pre_body,
        grid=(nb,),
        in_specs=[
            pl.BlockSpec((BE, H), lambda i: (i, 0)),
            pl.BlockSpec((6, BE), lambda i: (0, i)),
            pl.BlockSpec((6, H), lambda i: (0, 0)),
            pl.BlockSpec((H, H), lambda i: (0, 0)),
            pl.BlockSpec((1, H), lambda i: (0, 0)),
            pl.BlockSpec((H, H), lambda i: (0, 0)),
            pl.BlockSpec((1, H), lambda i: (0, 0)),
        ],
        out_specs=[
            pl.BlockSpec((BE, H), lambda i: (i, 0)),
            pl.BlockSpec((BE, H), lambda i: (i, 0)),
        ],
        out_shape=[
            jax.ShapeDtypeStruct((E, H), jnp.float32),
            jax.ShapeDtypeStruct((E, H), jnp.float32),
        ],
    )(x, rbf.T, W_rbf, W_kj, b_kj.reshape(1, H), W_ji, b_ji.reshape(1, H))


BT = 3200


def _triplet_body(g_ref, sbft_ref, wsbf_ref, wbil_ref, out_ref):
    st = jax.lax.dot_general(sbft_ref[...], wsbf_ref[...],
                             (((0,), (0,)), ((), ())),
                             preferred_element_type=jnp.float32)
    g = g_ref[...]
    acc = jnp.zeros((BT, H), jnp.float32)
    for j in range(NB):
        acc = acc + jnp.dot(g * st[:, j:j + 1], wbil_ref[j],
                            preferred_element_type=jnp.float32)
    out_ref[...] = acc


def _triplet(gathered, sbf, W_sbf, Wbil_t):
    nb = T // BT
    return pl.pallas_call(
        _triplet_body,
        grid=(nb,),
        in_specs=[
            pl.BlockSpec((BT, H), lambda i: (i, 0)),
            pl.BlockSpec((42, BT), lambda i: (0, i)),
            pl.BlockSpec((42, NB), lambda i: (0, 0)),
            pl.BlockSpec((NB, H, H), lambda i: (0, 0, 0)),
        ],
        out_specs=pl.BlockSpec((BT, H), lambda i: (i, 0)),
        out_shape=jax.ShapeDtypeStruct((T, H), jnp.float32),
    )(gathered, sbf.T, W_sbf, Wbil_t)


def _post_body(x_ref, xji_ref, seg_ref,
               w1_ref, b1_ref, w2_ref, b2_ref,
               wl_ref, bl_ref,
               w3_ref, b3_ref, w4_ref, b4_ref,
               w5_ref, b5_ref, w6_ref, b6_ref,
               out_ref):
    def mm(a, w_ref, b_ref):
        r = jnp.dot(a, w_ref[...], preferred_element_type=jnp.float32) + b_ref[...]
        return r * jax.nn.sigmoid(r)

    h = xji_ref[...] + seg_ref[...]
    h = h + mm(mm(h, w1_ref, b1_ref), w2_ref, b2_ref)
    h = mm(h, wl_ref, bl_ref) + x_ref[...]
    h = h + mm(mm(h, w3_ref, b3_ref), w4_ref, b4_ref)
    h = h + mm(mm(h, w5_ref, b5_ref), w6_ref, b6_ref)
    out_ref[...] = h


def _post(x, x_ji, seg, rb0_w1, rb0_b1, rb0_w2, rb0_b2, W_lin, b_lin,
          ra0_w1, ra0_b1, ra0_w2, ra0_b2, ra1_w1, ra1_b1, ra1_w2, ra1_b2):
    nb = E // BE
    wspec = pl.BlockSpec((H, H), lambda i: (0, 0))
    bspec = pl.BlockSpec((1, H), lambda i: (0, 0))
    return pl.pallas_call(
        _post_body,
        grid=(nb,),
        in_specs=[pl.BlockSpec((BE, H), lambda i: (i, 0))] * 3 +
                 [wspec, bspec] * 7,
        out_specs=pl.BlockSpec((BE, H), lambda i: (i, 0)),
        out_shape=jax.ShapeDtypeStruct((E, H), jnp.float32),
    )(x, x_ji, seg,
      rb0_w1, rb0_b1.reshape(1, H), rb0_w2, rb0_b2.reshape(1, H),
      W_lin, b_lin.reshape(1, H),
      ra0_w1, ra0_b1.reshape(1, H), ra0_w2, ra0_b2.reshape(1, H),
      ra1_w1, ra1_b1.reshape(1, H), ra1_w2, ra1_b2.reshape(1, H))


TPW = T // NW
GB = 80
NGB = TPW // GB


def _sc_gather_body(table_hbm, idx_hbm, out_hbm, idx_v, rows_v, gsem, osem):
    c = lax.axis_index("c")
    s = lax.axis_index("s")
    wid = s * NC + c
    base = wid * TPW
    pltpu.sync_copy(idx_hbm.at[pl.ds(base * 1, TPW)], idx_v)

    def g_desc(b, slot):
        return pltpu.make_async_copy(
            table_hbm.at[idx_v.at[pl.ds(b * GB, GB)]],
            rows_v.at[slot], gsem.at[slot])

    def o_desc(b, slot):
        return pltpu.make_async_copy(
            rows_v.at[slot], out_hbm.at[pl.ds(base + b * GB, GB)],
            osem.at[slot])

    g_desc(0, 0).start()

    def body(b, _):
        slot = lax.rem(b, 2)
        nslot = 1 - slot
        g_desc(b, slot).wait()

        @pl.when(b >= 1)
        def _():
            o_desc(b - 1, nslot).wait()

        @pl.when(b < NGB - 1)
        def _():
            g_desc(b + 1, nslot).start()

        o_desc(b, slot).start()
        return 0

    lax.fori_loop(0, NGB, body, 0)
    o_desc(NGB - 1, (NGB - 1) % 2).wait()


def _sc_gather(table, idx):
    mesh = plsc.VectorSubcoreMesh(core_axis_name="c", subcore_axis_name="s")
    f = pl.kernel(
        _sc_gather_body,
        out_type=jax.ShapeDtypeStruct((T, H), jnp.float32),
        mesh=mesh,
        scratch_types=[
            pltpu.VMEM((TPW,), jnp.int32),
            pltpu.VMEM((2, GB, H), jnp.float32),
            pltpu.SemaphoreType.DMA((2,)),
            pltpu.SemaphoreType.DMA((2,)),
        ],
    )
    return f(table, idx)


EPC = E // NC
ACC = EPC + NSUB
TPS = T // NSUB
SB = 80
RB = 800
NRB = TPS // RB
SPB = RB // SB
NH = H // L
ZR = ACC // NSUB
ZC = 125
WR = EPC // NSUB
IC = 2000
NIC = TPS // IC


def _sc_scatter_body(outt_hbm, idx_hbm, seg_hbm, idxc, rel2d, rows_v, zbuf,
                     acc, rsem, ssem):
    c = lax.axis_index("c")
    s = lax.axis_index("s")
    t0 = s * TPS
    lo = c * EPC
    trash = EPC + s

    def zb(i, _):
        zbuf[i, :] = jnp.zeros((L,), jnp.float32)
        return 0
    lax.fori_loop(0, ZC, zb, 0)

    def relchunk(ci, _):
        pltpu.sync_copy(idx_hbm.at[pl.ds(t0 + ci * IC, IC)], idxc)

        def relbody(i, _):
            v = idxc[pl.ds(i * L, L)]
            rel = v - lo
            inb = (rel >= 0) & (rel < EPC)
            relv = jnp.where(inb, rel, trash)
            rel2d[ci * (IC // SB) + lax.div(i, 5),
                  pl.ds(lax.rem(i, 5) * L, L)] = relv
            return 0
        lax.fori_loop(0, IC // L, relbody, 0)
        return 0
    lax.fori_loop(0, NIC, relchunk, 0)

    def r_desc(rb, slot, h):
        return pltpu.make_async_copy(
            outt_hbm.at[pl.ds(t0 + rb * RB, RB), pl.ds(h * L, L)],
            rows_v.at[slot], rsem.at[slot])

    def s_desc(rb, k, slot):
        return pltpu.make_async_copy(
            rows_v.at[slot, pl.ds(k * SB, SB)],
            acc.at[rel2d.at[rb * SPB + k]], ssem.at[slot])

    def one_pass(h, _):
        for z in range(40):
            pltpu.sync_copy(zbuf, acc.at[pl.ds(s * ZR + z * ZC, ZC)])
        pltpu.sync_copy(zbuf.at[pl.ds(0, 1)], acc.at[pl.ds(s * ZR + 40 * ZC, 1)])
        plsc.subcore_barrier()

        r_desc(0, 0, h).start()

        def body(rb, _):
            slot = lax.rem(rb, 2)
            nslot = 1 - slot
            r_desc(rb, slot, h).wait()

            @pl.when(rb < NRB - 1)
            def _():
                r_desc(rb + 1, nslot, h).start()

            for k in range(SPB):
                pltpu.async_copy(rows_v.at[slot, pl.ds(k * SB, SB)],
                                 acc.at[rel2d.at[rb * SPB + k]],
                                 ssem.at[slot], add=True)
            for k in range(SPB):
                s_desc(rb, k, slot).wait()
            return 0

        lax.fori_loop(0, NRB, body, 0)
        plsc.subcore_barrier()

        pltpu.sync_copy(
            acc.at[pl.ds(s * WR, WR)],
            seg_hbm.at[pl.ds(lo + s * WR, WR), pl.ds(h * L, L)])
        plsc.subcore_barrier()
        return 0

    lax.fori_loop(0, NH, one_pass, 0)


def _sc_scatter(out_t, idx):
    mesh = plsc.VectorSubcoreMesh(core_axis_name="c", subcore_axis_name="s")
    f = pl.kernel(
        _sc_scatter_body,
        out_type=jax.ShapeDtypeStruct((E, H), jnp.float32),
        mesh=mesh,
        compiler_params=pltpu.CompilerParams(use_tc_tiling_on_sc=False),
        scratch_types=[
            pltpu.VMEM((IC,), jnp.int32),
            pltpu.VMEM((TPS // SB, SB), jnp.int32),
            pltpu.VMEM((2, RB, L), jnp.float32),
            pltpu.VMEM((ZC, L), jnp.float32),
            pltpu.MemorySpace.VMEM_SHARED((ACC, L), jnp.float32),
            pltpu.SemaphoreType.DMA((2,)),
            pltpu.SemaphoreType.DMA((2,)),
        ],
    )
    return f(out_t, idx)


def kernel(x, rbf, sbf, idx_kj, idx_ji, W_rbf, W_sbf, W_kj, b_kj, W_ji, b_ji,
           Wbil, rb0_w1, rb0_b1, rb0_w2, rb0_b2, W_lin, b_lin,
           ra0_w1, ra0_b1, ra0_w2, ra0_b2, ra1_w1, ra1_b1, ra1_w2, ra1_b2):
    x_ji, x_kj2 = _edge_pre(x, rbf, W_rbf, W_kj, b_kj, W_ji, b_ji)
    gathered = _sc_gather(x_kj2, idx_kj)
    Wbil_t = jnp.transpose(Wbil, (1, 2, 0))
    out_t = _triplet(gathered, sbf, W_sbf, Wbil_t)
    seg = _sc_scatter(out_t, idx_ji)
    return _post(x, x_ji, seg, rb0_w1, rb0_b1, rb0_w2, rb0_b2, W_lin, b_lin,
                 ra0_w1, ra0_b1, ra0_w2, ra0_b2, ra1_w1, ra1_b1, ra1_w2, ra1_b2)


# speedup vs baseline: 1.1208x; 1.0207x over previous
import functools

import jax
import jax.numpy as jnp
from jax import lax
from jax.experimental import pallas as pl
from jax.experimental.pallas import tpu as pltpu
from jax.experimental.pallas import tpu_sc as plsc

H = 128
NB = 8
E = 160000
T = 320000

NC = 2
NSUB = 16
NW = NC * NSUB
L = 16

BE = 3200


def _edge_pre_body(x_ref, rbft_ref, wrbf_ref, wkj_ref, bkj_ref, wji_ref,
                   bji_ref, xji_ref, xkj_ref):
    xb = x_ref[...]
    ji = jnp.dot(xb, wji_ref[...], preferred_element_type=jnp.float32) + bji_ref[...]
    xji_ref[...] = ji * jax.nn.sigmoid(ji)
    kj = jnp.dot(xb, wkj_ref[...], preferred_element_type=jnp.float32) + bkj_ref[...]
    rt = jax.lax.dot_general(rbft_ref[...], wrbf_ref[...],
                             (((0,), (0,)), ((), ())),
                             preferred_element_type=jnp.float32)
    xkj_ref[...] = kj * jax.nn.sigmoid(kj) * rt


def _edge_pre(x, rbf, W_rbf, W_kj, b_kj, W_ji, b_ji):
    nb = E // BE
    return pl.pallas_call(
        _edge_pre_body,
        grid=(nb,),
        in_specs=[
            pl.BlockSpec((BE, H), lambda i: (i, 0)),
            pl.BlockSpec((6, BE), lambda i: (0, i)),
            pl.BlockSpec((6, H), lambda i: (0, 0)),
            pl.BlockSpec((H, H), lambda i: (0, 0)),
            pl.BlockSpec((1, H), lambda i: (0, 0)),
            pl.BlockSpec((H, H), lambda i: (0, 0)),
            pl.BlockSpec((1, H), lambda i: (0, 0)),
        ],
        out_specs=[
            pl.BlockSpec((BE, H), lambda i: (i, 0)),
            pl.BlockSpec((BE, H), lambda i: (i, 0)),
        ],
        out_shape=[
            jax.ShapeDtypeStruct((E, H), jnp.float32),
            jax.ShapeDtypeStruct((E, H), jnp.float32),
        ],
    )(x, rbf.T, W_rbf, W_kj, b_kj.reshape(1, H), W_ji, b_ji.reshape(1, H))


BT = 3200


def _triplet_body(g_ref, sbft_ref, wsbf_ref, wbil_ref, out_ref):
    st = jax.lax.dot_general(sbft_ref[...], wsbf_ref[...],
                             (((0,), (0,)), ((), ())),
                             preferred_element_type=jnp.float32)
    g = g_ref[...]
    acc = jnp.zeros((BT, H), jnp.float32)
    for j in range(NB):
        acc = acc + jnp.dot(g * st[:, j:j + 1], wbil_ref[j],
                            preferred_element_type=jnp.float32)
    out_ref[...] = acc


def _triplet(gathered, sbf, W_sbf, Wbil_t):
    nb = T // BT
    return pl.pallas_call(
        _triplet_body,
        grid=(nb,),
        in_specs=[
            pl.BlockSpec((BT, H), lambda i: (i, 0)),
            pl.BlockSpec((42, BT), lambda i: (0, i)),
            pl.BlockSpec((42, NB), lambda i: (0, 0)),
            pl.BlockSpec((NB, H, H), lambda i: (0, 0, 0)),
        ],
        out_specs=pl.BlockSpec((BT, H), lambda i: (i, 0)),
        out_shape=jax.ShapeDtypeStruct((T, H), jnp.float32),
    )(gathered, sbf.T, W_sbf, Wbil_t)


def _post_body(x_ref, xji_ref, seg_ref,
               w1_ref, b1_ref, w2_ref, b2_ref,
               wl_ref, bl_ref,
               w3_ref, b3_ref, w4_ref, b4_ref,
               w5_ref, b5_ref, w6_ref, b6_ref,
               out_ref):
    def mm(a, w_ref, b_ref):
        r = jnp.dot(a, w_ref[...], preferred_element_type=jnp.float32) + b_ref[...]
        return r * jax.nn.sigmoid(r)

    h = xji_ref[...] + seg_ref[...]
    h = h + mm(mm(h, w1_ref, b1_ref), w2_ref, b2_ref)
    h = mm(h, wl_ref, bl_ref) + x_ref[...]
    h = h + mm(mm(h, w3_ref, b3_ref), w4_ref, b4_ref)
    h = h + mm(mm(h, w5_ref, b5_ref), w6_ref, b6_ref)
    out_ref[...] = h


def _post(x, x_ji, seg, rb0_w1, rb0_b1, rb0_w2, rb0_b2, W_lin, b_lin,
          ra0_w1, ra0_b1, ra0_w2, ra0_b2, ra1_w1, ra1_b1, ra1_w2, ra1_b2):
    nb = E // BE
    wspec = pl.BlockSpec((H, H), lambda i: (0, 0))
    bspec = pl.BlockSpec((1, H), lambda i: (0, 0))
    return pl.pallas_call(
        _post_body,
        grid=(nb,),
        in_specs=[pl.BlockSpec((BE, H), lambda i: (i, 0))] * 3 +
                 [wspec, bspec] * 7,
        out_specs=pl.BlockSpec((BE, H), lambda i: (i, 0)),
        out_shape=jax.ShapeDtypeStruct((E, H), jnp.float32),
    )(x, x_ji, seg,
      rb0_w1, rb0_b1.reshape(1, H), rb0_w2, rb0_b2.reshape(1, H),
      W_lin, b_lin.reshape(1, H),
      ra0_w1, ra0_b1.reshape(1, H), ra0_w2, ra0_b2.reshape(1, H),
      ra1_w1, ra1_b1.reshape(1, H), ra1_w2, ra1_b2.reshape(1, H))


TPW = T // NW
GB = 80
NGB = TPW // GB


def _sc_gather_body(table_hbm, idx_hbm, out_hbm, idx_v, rows_v, gsem, osem):
    c = lax.axis_index("c")
    s = lax.axis_index("s")
    wid = s * NC + c
    base = wid * TPW
    pltpu.sync_copy(idx_hbm.at[pl.ds(base * 1, TPW)], idx_v)

    def g_desc(b, slot):
        return pltpu.make_async_copy(
            table_hbm.at[idx_v.at[pl.ds(b * GB, GB)]],
            rows_v.at[slot], gsem.at[slot])

    def o_desc(b, slot):
        return pltpu.make_async_copy(
            rows_v.at[slot], out_hbm.at[pl.ds(base + b * GB, GB)],
            osem.at[slot])

    g_desc(0, 0).start()

    def body(b, _):
        slot = lax.rem(b, 2)
        nslot = 1 - slot
        g_desc(b, slot).wait()

        @pl.when(b >= 1)
        def _():
            o_desc(b - 1, nslot).wait()

        @pl.when(b < NGB - 1)
        def _():
            g_desc(b + 1, nslot).start()

        o_desc(b, slot).start()
        return 0

    lax.fori_loop(0, NGB, body, 0)
    o_desc(NGB - 1, (NGB - 1) % 2).wait()


def _sc_gather(table, idx):
    mesh = plsc.VectorSubcoreMesh(core_axis_name="c", subcore_axis_name="s")
    f = pl.kernel(
        _sc_gather_body,
        out_type=jax.ShapeDtypeStruct((T, H), jnp.float32),
        mesh=mesh,
        scratch_types=[
            pltpu.VMEM((TPW,), jnp.int32),
            pltpu.VMEM((2, GB, H), jnp.float32),
            pltpu.SemaphoreType.DMA((2,)),
            pltpu.SemaphoreType.DMA((2,)),
        ],
    )
    return f(table, idx)


EPC = E // NC
ACC = EPC + NSUB
TPS = T // NSUB
SB = 80
RB = 800
NRB = TPS // RB
SPB = RB // SB
NH = H // L
ZR = ACC // NSUB
ZC = 125
WR = EPC // NSUB
IC = 2000
NIC = TPS // IC


def _sc_scatter_body(outt_hbm, idx_hbm, seg_hbm, idxc, rel2d, rows_v, zbuf,
                     acc, rsem, ssem, zsem):
    c = lax.axis_index("c")
    s = lax.axis_index("s")
    t0 = s * TPS
    lo = c * EPC
    trash = EPC + s

    def zb(i, _):
        zbuf[i, :] = jnp.zeros((L,), jnp.float32)
        return 0
    lax.fori_loop(0, ZC, zb, 0)

    def relchunk(ci, _):
        pltpu.sync_copy(idx_hbm.at[pl.ds(t0 + ci * IC, IC)], idxc)

        def relbody(i, _):
            v = idxc[pl.ds(i * L, L)]
            rel = v - lo
            inb = (rel >= 0) & (rel < EPC)
            relv = jnp.where(inb, rel, trash)
            rel2d[ci * (IC // SB) + lax.div(i, 5),
                  pl.ds(lax.rem(i, 5) * L, L)] = relv
            return 0
        lax.fori_loop(0, IC // L, relbody, 0)
        return 0
    lax.fori_loop(0, NIC, relchunk, 0)

    def r_desc(rb, slot, h):
        return pltpu.make_async_copy(
            outt_hbm.at[pl.ds(t0 + rb * RB, RB), pl.ds(h * L, L)],
            rows_v.at[slot], rsem.at[slot])

    def s_desc(rb, k, slot):
        return pltpu.make_async_copy(
            rows_v.at[slot, pl.ds(k * SB, SB)],
            acc.at[rel2d.at[rb * SPB + k]], ssem.at[slot])

    def z_desc(z):
        return pltpu.make_async_copy(
            zbuf, acc.at[pl.ds(s * ZR + z * ZC, ZC)], zsem)

    def z1_desc():
        return pltpu.make_async_copy(
            zbuf.at[pl.ds(0, 1)], acc.at[pl.ds(s * ZR + 40 * ZC, 1)], zsem)

    def one_pass(h, _):
        r_desc(0, 0, h).start()
        for z in range(40):
            z_desc(z).start()
        z1_desc().start()
        for z in range(40):
            z_desc(z).wait()
        z1_desc().wait()
        plsc.subcore_barrier()

        def body(rb, _):
            slot = lax.rem(rb, 2)
            nslot = 1 - slot
            r_desc(rb, slot, h).wait()

            @pl.when(rb < NRB - 1)
            def _():
                r_desc(rb + 1, nslot, h).start()

            for k in range(SPB):
                pltpu.async_copy(rows_v.at[slot, pl.ds(k * SB, SB)],
                                 acc.at[rel2d.at[rb * SPB + k]],
                                 ssem.at[slot], add=True)
            for k in range(SPB):
                s_desc(rb, k, slot).wait()
            return 0

        lax.fori_loop(0, NRB, body, 0)
        plsc.subcore_barrier()

        pltpu.sync_copy(
            acc.at[pl.ds(s * WR, WR)],
            seg_hbm.at[pl.ds(lo + s * WR, WR), pl.ds(h * L, L)])
        plsc.subcore_barrier()
        return 0

    lax.fori_loop(0, NH, one_pass, 0)


def _sc_scatter(out_t, idx):
    mesh = plsc.VectorSubcoreMesh(core_axis_name="c", subcore_axis_name="s")
    f = pl.kernel(
        _sc_scatter_body,
        out_type=jax.ShapeDtypeStruct((E, H), jnp.float32),
        mesh=mesh,
        compiler_params=pltpu.CompilerParams(use_tc_tiling_on_sc=False),
        scratch_types=[
            pltpu.VMEM((IC,), jnp.int32),
            pltpu.VMEM((TPS // SB, SB), jnp.int32),
            pltpu.VMEM((2, RB, L), jnp.float32),
            pltpu.VMEM((ZC, L), jnp.float32),
            pltpu.MemorySpace.VMEM_SHARED((ACC, L), jnp.float32),
            pltpu.SemaphoreType.DMA((2,)),
            pltpu.SemaphoreType.DMA((2,)),
            pltpu.SemaphoreType.DMA,
        ],
    )
    return f(out_t, idx)


def kernel(x, rbf, sbf, idx_kj, idx_ji, W_rbf, W_sbf, W_kj, b_kj, W_ji, b_ji,
           Wbil, rb0_w1, rb0_b1, rb0_w2, rb0_b2, W_lin, b_lin,
           ra0_w1, ra0_b1, ra0_w2, ra0_b2, ra1_w1, ra1_b1, ra1_w2, ra1_b2):
    x_ji, x_kj2 = _edge_pre(x, rbf, W_rbf, W_kj, b_kj, W_ji, b_ji)
    gathered = _sc_gather(x_kj2, idx_kj)
    Wbil_t = jnp.transpose(Wbil, (1, 2, 0))
    out_t = _triplet(gathered, sbf, W_sbf, Wbil_t)
    seg = _sc_scatter(out_t, idx_ji)
    return _post(x, x_ji, seg, rb0_w1, rb0_b1, rb0_w2, rb0_b2, W_lin, b_lin,
                 ra0_w1, ra0_b1, ra0_w2, ra0_b2, ra1_w1, ra1_b1, ra1_w2, ra1_b2)


# speedup vs baseline: 1.1668x; 1.0410x over previous
import functools

import jax
import jax.numpy as jnp
from jax import lax
from jax.experimental import pallas as pl
from jax.experimental.pallas import tpu as pltpu
from jax.experimental.pallas import tpu_sc as plsc

H = 128
NB = 8
E = 160000
T = 320000

NC = 2
NSUB = 16
NW = NC * NSUB
L = 16

BE = 3200


def _edge_pre_body(x_ref, rbft_ref, wrbf_ref, wkj_ref, bkj_ref, wji_ref,
                   bji_ref, xji_ref, xkj_ref):
    xb = x_ref[...]
    ji = jnp.dot(xb, wji_ref[...], preferred_element_type=jnp.float32) + bji_ref[...]
    xji_ref[...] = ji * jax.nn.sigmoid(ji)
    kj = jnp.dot(xb, wkj_ref[...], preferred_element_type=jnp.float32) + bkj_ref[...]
    rt = jax.lax.dot_general(rbft_ref[...], wrbf_ref[...],
                             (((0,), (0,)), ((), ())),
                             preferred_element_type=jnp.float32)
    xkj_ref[...] = kj * jax.nn.sigmoid(kj) * rt


def _edge_pre(x, rbf, W_rbf, W_kj, b_kj, W_ji, b_ji):
    nb = E // BE
    return pl.pallas_call(
        _edge_pre_body,
        grid=(nb,),
        in_specs=[
            pl.BlockSpec((BE, H), lambda i: (i, 0)),
            pl.BlockSpec((6, BE), lambda i: (0, i)),
            pl.BlockSpec((6, H), lambda i: (0, 0)),
            pl.BlockSpec((H, H), lambda i: (0, 0)),
            pl.BlockSpec((1, H), lambda i: (0, 0)),
            pl.BlockSpec((H, H), lambda i: (0, 0)),
            pl.BlockSpec((1, H), lambda i: (0, 0)),
        ],
        out_specs=[
            pl.BlockSpec((BE, H), lambda i: (i, 0)),
            pl.BlockSpec((BE, H), lambda i: (i, 0)),
        ],
        out_shape=[
            jax.ShapeDtypeStruct((E, H), jnp.float32),
            jax.ShapeDtypeStruct((E, H), jnp.float32),
        ],
    )(x, rbf.T, W_rbf, W_kj, b_kj.reshape(1, H), W_ji, b_ji.reshape(1, H))


BT = 3200


def _triplet_body(g_ref, sbft_ref, wsbf_ref, wbil_ref, out_ref):
    st = jax.lax.dot_general(sbft_ref[...], wsbf_ref[...],
                             (((0,), (0,)), ((), ())),
                             preferred_element_type=jnp.float32)
    g = g_ref[...]
    acc = jnp.zeros((BT, H), jnp.float32)
    for j in range(NB):
        acc = acc + jnp.dot(g * st[:, j:j + 1], wbil_ref[j],
                            preferred_element_type=jnp.float32)
    out_ref[...] = acc


def _triplet(gathered, sbf, W_sbf, Wbil_t):
    nb = T // BT
    return pl.pallas_call(
        _triplet_body,
        grid=(nb,),
        in_specs=[
            pl.BlockSpec((BT, H), lambda i: (i, 0)),
            pl.BlockSpec((42, BT), lambda i: (0, i)),
            pl.BlockSpec((42, NB), lambda i: (0, 0)),
            pl.BlockSpec((NB, H, H), lambda i: (0, 0, 0)),
        ],
        out_specs=pl.BlockSpec((BT, H), lambda i: (i, 0)),
        out_shape=jax.ShapeDtypeStruct((T, H), jnp.float32),
    )(gathered, sbf.T, W_sbf, Wbil_t)


def _post_body(x_ref, xji_ref, seg_ref,
               w1_ref, b1_ref, w2_ref, b2_ref,
               wl_ref, bl_ref,
               w3_ref, b3_ref, w4_ref, b4_ref,
               w5_ref, b5_ref, w6_ref, b6_ref,
               out_ref):
    def mm(a, w_ref, b_ref):
        r = jnp.dot(a, w_ref[...], preferred_element_type=jnp.float32) + b_ref[...]
        return r * jax.nn.sigmoid(r)

    h = xji_ref[...] + seg_ref[...]
    h = h + mm(mm(h, w1_ref, b1_ref), w2_ref, b2_ref)
    h = mm(h, wl_ref, bl_ref) + x_ref[...]
    h = h + mm(mm(h, w3_ref, b3_ref), w4_ref, b4_ref)
    h = h + mm(mm(h, w5_ref, b5_ref), w6_ref, b6_ref)
    out_ref[...] = h


def _post(x, x_ji, seg, rb0_w1, rb0_b1, rb0_w2, rb0_b2, W_lin, b_lin,
          ra0_w1, ra0_b1, ra0_w2, ra0_b2, ra1_w1, ra1_b1, ra1_w2, ra1_b2):
    nb = E // BE
    wspec = pl.BlockSpec((H, H), lambda i: (0, 0))
    bspec = pl.BlockSpec((1, H), lambda i: (0, 0))
    return pl.pallas_call(
        _post_body,
        grid=(nb,),
        in_specs=[pl.BlockSpec((BE, H), lambda i: (i, 0))] * 3 +
                 [wspec, bspec] * 7,
        out_specs=pl.BlockSpec((BE, H), lambda i: (i, 0)),
        out_shape=jax.ShapeDtypeStruct((E, H), jnp.float32),
    )(x, x_ji, seg,
      rb0_w1, rb0_b1.reshape(1, H), rb0_w2, rb0_b2.reshape(1, H),
      W_lin, b_lin.reshape(1, H),
      ra0_w1, ra0_b1.reshape(1, H), ra0_w2, ra0_b2.reshape(1, H),
      ra1_w1, ra1_b1.reshape(1, H), ra1_w2, ra1_b2.reshape(1, H))


TPW = T // NW
GB = 80
NGB = TPW // GB


def _sc_gather_body(table_hbm, idx_hbm, out_hbm, idx_v, rows_v, gsem, osem):
    c = lax.axis_index("c")
    s = lax.axis_index("s")
    wid = s * NC + c
    base = wid * TPW
    pltpu.sync_copy(idx_hbm.at[pl.ds(base * 1, TPW)], idx_v)

    def g_desc(b, slot):
        return pltpu.make_async_copy(
            table_hbm.at[idx_v.at[pl.ds(b * GB, GB)]],
            rows_v.at[slot], gsem.at[slot])

    def o_desc(b, slot):
        return pltpu.make_async_copy(
            rows_v.at[slot], out_hbm.at[pl.ds(base + b * GB, GB)],
            osem.at[slot])

    for r in range(3):
        g_desc(r, r).start()

    def body(b, _):
        slot = lax.rem(b, 4)
        g_desc(b, slot).wait()
        o_desc(b, slot).start()

        @pl.when(b >= 1)
        def _():
            o_desc(b - 1, lax.rem(b - 1, 4)).wait()

        @pl.when(b + 3 < NGB)
        def _():
            g_desc(b + 3, lax.rem(b + 3, 4)).start()

        return 0

    lax.fori_loop(0, NGB, body, 0)
    o_desc(NGB - 1, (NGB - 1) % 4).wait()


def _sc_gather(table, idx):
    mesh = plsc.VectorSubcoreMesh(core_axis_name="c", subcore_axis_name="s")
    f = pl.kernel(
        _sc_gather_body,
        out_type=jax.ShapeDtypeStruct((T, H), jnp.float32),
        mesh=mesh,
        scratch_types=[
            pltpu.VMEM((TPW,), jnp.int32),
            pltpu.VMEM((4, GB, H), jnp.float32),
            pltpu.SemaphoreType.DMA((4,)),
            pltpu.SemaphoreType.DMA((4,)),
        ],
    )
    return f(table, idx)


EPC = E // NC
ACC = EPC + NSUB
TPS = T // NSUB
SB = 80
RB = 800
NRB = TPS // RB
SPB = RB // SB
NH = H // L
ZR = ACC // NSUB
ZC = 125
WR = EPC // NSUB
IC = 2000
NIC = TPS // IC


def _sc_scatter_body(outt_hbm, idx_hbm, seg_hbm, idxc, rel2d, rows_v, zbuf,
                     acc, rsem, ssem, zsem):
    c = lax.axis_index("c")
    s = lax.axis_index("s")
    t0 = s * TPS
    lo = c * EPC
    trash = EPC + s

    def zb(i, _):
        zbuf[i, :] = jnp.zeros((L,), jnp.float32)
        return 0
    lax.fori_loop(0, ZC, zb, 0)

    def relchunk(ci, _):
        pltpu.sync_copy(idx_hbm.at[pl.ds(t0 + ci * IC, IC)], idxc)

        def relbody(i, _):
            v = idxc[pl.ds(i * L, L)]
            rel = v - lo
            inb = (rel >= 0) & (rel < EPC)
            relv = jnp.where(inb, rel, trash)
            rel2d[ci * (IC // SB) + lax.div(i, 5),
                  pl.ds(lax.rem(i, 5) * L, L)] = relv
            return 0
        lax.fori_loop(0, IC // L, relbody, 0)
        return 0
    lax.fori_loop(0, NIC, relchunk, 0)

    def r_desc(rb, slot, h):
        return pltpu.make_async_copy(
            outt_hbm.at[pl.ds(t0 + rb * RB, RB), pl.ds(h * L, L)],
            rows_v.at[slot], rsem.at[slot])

    def s_desc(rb, k, slot):
        return pltpu.make_async_copy(
            rows_v.at[slot, pl.ds(k * SB, SB)],
            acc.at[rel2d.at[rb * SPB + k]], ssem.at[slot])

    def z_desc(z):
        return pltpu.make_async_copy(
            zbuf, acc.at[pl.ds(s * ZR + z * ZC, ZC)], zsem)

    def z1_desc():
        return pltpu.make_async_copy(
            zbuf.at[pl.ds(0, 1)], acc.at[pl.ds(s * ZR + 40 * ZC, 1)], zsem)

    def one_pass(h, _):
        r_desc(0, 0, h).start()
        for z in range(40):
            z_desc(z).start()
        z1_desc().start()
        for z in range(40):
            z_desc(z).wait()
        z1_desc().wait()
        plsc.subcore_barrier()

        def body(rb, _):
            slot = lax.rem(rb, 2)
            nslot = 1 - slot
            r_desc(rb, slot, h).wait()

            @pl.when(rb < NRB - 1)
            def _():
                r_desc(rb + 1, nslot, h).start()

            for k in range(SPB):
                pltpu.async_copy(rows_v.at[slot, pl.ds(k * SB, SB)],
                                 acc.at[rel2d.at[rb * SPB + k]],
                                 ssem.at[slot], add=True)
            for k in range(SPB):
                s_desc(rb, k, slot).wait()
            return 0

        lax.fori_loop(0, NRB, body, 0)
        plsc.subcore_barrier()

        pltpu.sync_copy(
            acc.at[pl.ds(s * WR, WR)],
            seg_hbm.at[pl.ds(lo + s * WR, WR), pl.ds(h * L, L)])
        plsc.subcore_barrier()
        return 0

    lax.fori_loop(0, NH, one_pass, 0)


def _sc_scatter(out_t, idx):
    mesh = plsc.VectorSubcoreMesh(core_axis_name="c", subcore_axis_name="s")
    f = pl.kernel(
        _sc_scatter_body,
        out_type=jax.ShapeDtypeStruct((E, H), jnp.float32),
        mesh=mesh,
        compiler_params=pltpu.CompilerParams(use_tc_tiling_on_sc=False),
        scratch_types=[
            pltpu.VMEM((IC,), jnp.int32),
            pltpu.VMEM((TPS // SB, SB), jnp.int32),
            pltpu.VMEM((2, RB, L), jnp.float32),
            pltpu.VMEM((ZC, L), jnp.float32),
            pltpu.MemorySpace.VMEM_SHARED((ACC, L), jnp.float32),
            pltpu.SemaphoreType.DMA((2,)),
            pltpu.SemaphoreType.DMA((2,)),
            pltpu.SemaphoreType.DMA,
        ],
    )
    return f(out_t, idx)


def kernel(x, rbf, sbf, idx_kj, idx_ji, W_rbf, W_sbf, W_kj, b_kj, W_ji, b_ji,
           Wbil, rb0_w1, rb0_b1, rb0_w2, rb0_b2, W_lin, b_lin,
           ra0_w1, ra0_b1, ra0_w2, ra0_b2, ra1_w1, ra1_b1, ra1_w2, ra1_b2):
    x_ji, x_kj2 = _edge_pre(x, rbf, W_rbf, W_kj, b_kj, W_ji, b_ji)
    gathered = _sc_gather(x_kj2, idx_kj)
    Wbil_t = jnp.transpose(Wbil, (1, 2, 0))
    out_t = _triplet(gathered, sbf, W_sbf, Wbil_t)
    seg = _sc_scatter(out_t, idx_ji)
    return _post(x, x_ji, seg, rb0_w1, rb0_b1, rb0_w2, rb0_b2, W_lin, b_lin,
                 ra0_w1, ra0_b1, ra0_w2, ra0_b2, ra1_w1, ra1_b1, ra1_w2, ra1_b2)


# speedup vs baseline: 1.2069x; 1.0344x over previous
import functools

import jax
import jax.numpy as jnp
from jax import lax
from jax.experimental import pallas as pl
from jax.experimental.pallas import tpu as pltpu
from jax.experimental.pallas import tpu_sc as plsc

H = 128
NB = 8
E = 160000
T = 320000

NC = 2
NSUB = 16
NW = NC * NSUB
L = 16

BE = 3200


def _edge_pre_body(x_ref, rbft_ref, wrbf_ref, wkj_ref, bkj_ref, wji_ref,
                   bji_ref, xji_ref, xkj_ref):
    xb = x_ref[...]
    ji = jnp.dot(xb, wji_ref[...], preferred_element_type=jnp.float32) + bji_ref[...]
    xji_ref[...] = ji * jax.nn.sigmoid(ji)
    kj = jnp.dot(xb, wkj_ref[...], preferred_element_type=jnp.float32) + bkj_ref[...]
    rt = jax.lax.dot_general(rbft_ref[...], wrbf_ref[...],
                             (((0,), (0,)), ((), ())),
                             preferred_element_type=jnp.float32)
    xkj_ref[...] = kj * jax.nn.sigmoid(kj) * rt


def _edge_pre(x, rbf, W_rbf, W_kj, b_kj, W_ji, b_ji):
    nb = E // BE
    return pl.pallas_call(
        _edge_pre_body,
        grid=(nb,),
        in_specs=[
            pl.BlockSpec((BE, H), lambda i: (i, 0)),
            pl.BlockSpec((6, BE), lambda i: (0, i)),
            pl.BlockSpec((6, H), lambda i: (0, 0)),
            pl.BlockSpec((H, H), lambda i: (0, 0)),
            pl.BlockSpec((1, H), lambda i: (0, 0)),
            pl.BlockSpec((H, H), lambda i: (0, 0)),
            pl.BlockSpec((1, H), lambda i: (0, 0)),
        ],
        out_specs=[
            pl.BlockSpec((BE, H), lambda i: (i, 0)),
            pl.BlockSpec((BE, H), lambda i: (i, 0)),
        ],
        out_shape=[
            jax.ShapeDtypeStruct((E, H), jnp.float32),
            jax.ShapeDtypeStruct((E, H), jnp.float32),
        ],
    )(x, rbf.T, W_rbf, W_kj, b_kj.reshape(1, H), W_ji, b_ji.reshape(1, H))


BT = 3200


def _triplet_body(g_ref, sbft_ref, wsbf_ref, wbil_ref, out_ref):
    st = jax.lax.dot_general(sbft_ref[...], wsbf_ref[...],
                             (((0,), (0,)), ((), ())),
                             preferred_element_type=jnp.float32)
    g = g_ref[...]
    acc = jnp.zeros((BT, H), jnp.float32)
    for j in range(NB):
        acc = acc + jnp.dot(g * st[:, j:j + 1], wbil_ref[j],
                            preferred_element_type=jnp.float32)
    out_ref[...] = acc


NCH = 2
TC_ = T // NCH


def _triplet_chunk(gathered_c, sbf_t, W_sbf, Wbil_t, chunk, prev):
    nb = TC_ // BT
    base = chunk * nb
    specs = [
        pl.BlockSpec((BT, H), lambda i: (i, 0)),
        pl.BlockSpec((42, BT), lambda i: (0, base + i)),
        pl.BlockSpec((42, NB), lambda i: (0, 0)),
        pl.BlockSpec((NB, H, H), lambda i: (0, 0, 0)),
    ]
    args = [gathered_c, sbf_t, W_sbf, Wbil_t]
    aliases = {}
    if prev is not None:
        specs.append(pl.BlockSpec(memory_space=pl.ANY))
        args.append(prev)
        aliases = {4: 0}

    def body(*refs):
        _triplet_body(refs[0], refs[1], refs[2], refs[3], refs[-1])

    return pl.pallas_call(
        body,
        grid=(nb,),
        in_specs=specs,
        out_specs=pl.BlockSpec((BT, H), lambda i: (base + i, 0)),
        out_shape=jax.ShapeDtypeStruct((T, H), jnp.float32),
        input_output_aliases=aliases,
    )(*args)


def _post_body(x_ref, xji_ref, seg_ref,
               w1_ref, b1_ref, w2_ref, b2_ref,
               wl_ref, bl_ref,
               w3_ref, b3_ref, w4_ref, b4_ref,
               w5_ref, b5_ref, w6_ref, b6_ref,
               out_ref):
    def mm(a, w_ref, b_ref):
        r = jnp.dot(a, w_ref[...], preferred_element_type=jnp.float32) + b_ref[...]
        return r * jax.nn.sigmoid(r)

    h = xji_ref[...] + seg_ref[...]
    h = h + mm(mm(h, w1_ref, b1_ref), w2_ref, b2_ref)
    h = mm(h, wl_ref, bl_ref) + x_ref[...]
    h = h + mm(mm(h, w3_ref, b3_ref), w4_ref, b4_ref)
    h = h + mm(mm(h, w5_ref, b5_ref), w6_ref, b6_ref)
    out_ref[...] = h


def _post(x, x_ji, seg, rb0_w1, rb0_b1, rb0_w2, rb0_b2, W_lin, b_lin,
          ra0_w1, ra0_b1, ra0_w2, ra0_b2, ra1_w1, ra1_b1, ra1_w2, ra1_b2):
    nb = E // BE
    wspec = pl.BlockSpec((H, H), lambda i: (0, 0))
    bspec = pl.BlockSpec((1, H), lambda i: (0, 0))
    return pl.pallas_call(
        _post_body,
        grid=(nb,),
        in_specs=[pl.BlockSpec((BE, H), lambda i: (i, 0))] * 3 +
                 [wspec, bspec] * 7,
        out_specs=pl.BlockSpec((BE, H), lambda i: (i, 0)),
        out_shape=jax.ShapeDtypeStruct((E, H), jnp.float32),
    )(x, x_ji, seg,
      rb0_w1, rb0_b1.reshape(1, H), rb0_w2, rb0_b2.reshape(1, H),
      W_lin, b_lin.reshape(1, H),
      ra0_w1, ra0_b1.reshape(1, H), ra0_w2, ra0_b2.reshape(1, H),
      ra1_w1, ra1_b1.reshape(1, H), ra1_w2, ra1_b2.reshape(1, H))


TPW = 5000
GB = 40
NGB = TPW // GB


def _make_sc_gather_body(c0):
    def _sc_gather_body(table_hbm, idx_hbm, out_hbm, idx_v, rows_v, gsem, osem):
        c = lax.axis_index("c")
        s = lax.axis_index("s")
        wid = s * NC + c
        base = wid * TPW
        pltpu.sync_copy(idx_hbm.at[pl.ds(c0 + base, TPW)], idx_v)

        def g_desc(b, slot):
            return pltpu.make_async_copy(
                table_hbm.at[idx_v.at[pl.ds(b * GB, GB)]],
                rows_v.at[slot], gsem.at[slot])

        def o_desc(b, slot):
            return pltpu.make_async_copy(
                rows_v.at[slot], out_hbm.at[pl.ds(base + b * GB, GB)],
                osem.at[slot])

        for r in range(3):
            g_desc(r, r).start()

        def body(b, _):
            slot = lax.rem(b, 4)
            g_desc(b, slot).wait()
            o_desc(b, slot).start()

            @pl.when(b >= 1)
            def _():
                o_desc(b - 1, lax.rem(b - 1, 4)).wait()

            @pl.when(b + 3 < NGB)
            def _():
                g_desc(b + 3, lax.rem(b + 3, 4)).start()

            return 0

        lax.fori_loop(0, NGB, body, 0)
        o_desc(NGB - 1, (NGB - 1) % 4).wait()

    return _sc_gather_body


def _sc_gather(table, idx, c0):
    mesh = plsc.VectorSubcoreMesh(core_axis_name="c", subcore_axis_name="s")
    f = pl.kernel(
        _make_sc_gather_body(c0),
        out_type=jax.ShapeDtypeStruct((TC_, H), jnp.float32),
        mesh=mesh,
        scratch_types=[
            pltpu.VMEM((TPW,), jnp.int32),
            pltpu.VMEM((4, GB, H), jnp.float32),
            pltpu.SemaphoreType.DMA((4,)),
            pltpu.SemaphoreType.DMA((4,)),
        ],
    )
    return f(table, idx)


EPC = E // NC
ACC = EPC + NSUB
TPS = T // NSUB
SB = 80
RB = 800
NRB = TPS // RB
SPB = RB // SB
NH = H // L
ZR = ACC // NSUB
ZC = 125
WR = EPC // NSUB
IC = 2000
NIC = TPS // IC


def _sc_scatter_body(outt_hbm, idx_hbm, seg_hbm, idxc, rel2d, rows_v, zbuf,
                     acc, rsem, ssem, zsem):
    c = lax.axis_index("c")
    s = lax.axis_index("s")
    t0 = s * TPS
    lo = c * EPC
    trash = EPC + s

    def zb(i, _):
        zbuf[i, :] = jnp.zeros((L,), jnp.float32)
        return 0
    lax.fori_loop(0, ZC, zb, 0)

    def relchunk(ci, _):
        pltpu.sync_copy(idx_hbm.at[pl.ds(t0 + ci * IC, IC)], idxc)

        def relbody(i, _):
            v = idxc[pl.ds(i * L, L)]
            rel = v - lo
            inb = (rel >= 0) & (rel < EPC)
            relv = jnp.where(inb, rel, trash)
            rel2d[ci * (IC // SB) + lax.div(i, 5),
                  pl.ds(lax.rem(i, 5) * L, L)] = relv
            return 0
        lax.fori_loop(0, IC // L, relbody, 0)
        return 0
    lax.fori_loop(0, NIC, relchunk, 0)

    def r_desc(rb, slot, h):
        return pltpu.make_async_copy(
            outt_hbm.at[pl.ds(t0 + rb * RB, RB), pl.ds(h * L, L)],
            rows_v.at[slot], rsem.at[slot])

    def s_desc(rb, k, slot):
        return pltpu.make_async_copy(
            rows_v.at[slot, pl.ds(k * SB, SB)],
            acc.at[rel2d.at[rb * SPB + k]], ssem.at[slot])

    def z_desc(z):
        return pltpu.make_async_copy(
            zbuf, acc.at[pl.ds(s * ZR + z * ZC, ZC)], zsem)

    def z1_desc():
        return pltpu.make_async_copy(
            zbuf.at[pl.ds(0, 1)], acc.at[pl.ds(s * ZR + 40 * ZC, 1)], zsem)

    def one_pass(h, _):
        r_desc(0, 0, h).start()
        for z in range(40):
            z_desc(z).start()
        z1_desc().start()
        for z in range(40):
            z_desc(z).wait()
        z1_desc().wait()
        plsc.subcore_barrier()

        def body(rb, _):
            slot = lax.rem(rb, 2)
            nslot = 1 - slot
            r_desc(rb, slot, h).wait()

            @pl.when(rb < NRB - 1)
            def _():
                r_desc(rb + 1, nslot, h).start()

            for k in range(SPB):
                pltpu.async_copy(rows_v.at[slot, pl.ds(k * SB, SB)],
                                 acc.at[rel2d.at[rb * SPB + k]],
                                 ssem.at[slot], add=True)
            for k in range(SPB):
                s_desc(rb, k, slot).wait()
            return 0

        lax.fori_loop(0, NRB, body, 0)
        plsc.subcore_barrier()

        pltpu.sync_copy(
            acc.at[pl.ds(s * WR, WR)],
            seg_hbm.at[pl.ds(lo + s * WR, WR), pl.ds(h * L, L)])
        plsc.subcore_barrier()
        return 0

    lax.fori_loop(0, NH, one_pass, 0)


def _sc_scatter(out_t, idx):
    mesh = plsc.VectorSubcoreMesh(core_axis_name="c", subcore_axis_name="s")
    f = pl.kernel(
        _sc_scatter_body,
        out_type=jax.ShapeDtypeStruct((E, H), jnp.float32),
        mesh=mesh,
        compiler_params=pltpu.CompilerParams(use_tc_tiling_on_sc=False),
        scratch_types=[
            pltpu.VMEM((IC,), jnp.int32),
            pltpu.VMEM((TPS // SB, SB), jnp.int32),
            pltpu.VMEM((2, RB, L), jnp.float32),
            pltpu.VMEM((ZC, L), jnp.float32),
            pltpu.MemorySpace.VMEM_SHARED((ACC, L), jnp.float32),
            pltpu.SemaphoreType.DMA((2,)),
            pltpu.SemaphoreType.DMA((2,)),
            pltpu.SemaphoreType.DMA,
        ],
    )
    return f(out_t, idx)


def kernel(x, rbf, sbf, idx_kj, idx_ji, W_rbf, W_sbf, W_kj, b_kj, W_ji, b_ji,
           Wbil, rb0_w1, rb0_b1, rb0_w2, rb0_b2, W_lin, b_lin,
           ra0_w1, ra0_b1, ra0_w2, ra0_b2, ra1_w1, ra1_b1, ra1_w2, ra1_b2):
    x_ji, x_kj2 = _edge_pre(x, rbf, W_rbf, W_kj, b_kj, W_ji, b_ji)
    Wbil_t = jnp.transpose(Wbil, (1, 2, 0))
    sbf_t = sbf.T
    out_t = None
    gathered = [_sc_gather(x_kj2, idx_kj, ch * TC_) for ch in range(NCH)]
    for ch in range(NCH):
        out_t = _triplet_chunk(gathered[ch], sbf_t, W_sbf, Wbil_t, ch, out_t)
    seg = _sc_scatter(out_t, idx_ji)
    return _post(x, x_ji, seg, rb0_w1, rb0_b1, rb0_w2, rb0_b2, W_lin, b_lin,
                 ra0_w1, ra0_b1, ra0_w2, ra0_b2, ra1_w1, ra1_b1, ra1_w2, ra1_b2)
